# baseline (speedup 1.0000x reference)
import functools

import jax
import jax.numpy as jnp
from jax import lax
from jax.experimental import pallas as pl
from jax.experimental.pallas import tpu as pltpu
from jax.experimental.pallas import tpu_sc as plsc

_N = 10000
_E = 320000
_FIN = 128
_H = 16
_C = 10
_G = 64

_NC = 2
_NS = 16
_NW = _NC * _NS
_EPW = _E // _NW
_CH = 125
_NCH = _EPW // _CH
_NBUF = 8
_SLOT = 16
_NOUT = _NCH // _SLOT
_RPT = _N // _NS

_mesh = plsc.VectorSubcoreMesh(core_axis_name="c", subcore_axis_name="s")
_scp = pltpu.CompilerParams(use_tc_tiling_on_sc=False)


def _rsqrt16(d):
    i = lax.bitcast_convert_type(d, jnp.int32)
    i = jnp.int32(0x5F3759DF) - lax.shift_right_arithmetic(i, 1)
    y = lax.bitcast_convert_type(i, jnp.float32)
    for _ in range(2):
        y = y * (1.5 - 0.5 * d * y * y)
    return y


@functools.partial(
    pl.kernel,
    out_type=jax.ShapeDtypeStruct((_NC, _N, _H), jnp.float32),
    mesh=_mesh,
    scratch_types=[
        pltpu.VMEM((_NCH, _CH), jnp.int32),
        pltpu.VMEM((_CH, _H), jnp.float32),
        pltpu.VMEM_SHARED((_N, _H), jnp.float32),
    ],
    compiler_params=_scp,
)
def _sc_degree(ei_hbm, out_hbm, dstv, ones_v, acc):
    cid = lax.axis_index("c")
    sid = lax.axis_index("s")
    wid = sid * _NC + cid
    pltpu.sync_copy(ei_hbm.at[1, wid], dstv)
    ones16 = jnp.ones((_H,), jnp.float32)
    for r in range(_CH):
        ones_v[r, :] = ones16
    half = jnp.full((_H,), 0.5, jnp.float32)
    for r in range(_CH):
        ones_v[r, :] = half
    base = sid * _RPT
    for k in range(_RPT // _CH):
        pltpu.sync_copy(ones_v, acc.at[pl.ds(base + k * _CH, _CH)])
    for r in range(_CH):
        ones_v[r, :] = ones16
    plsc.subcore_barrier()

    def body(i, carry):
        pltpu.sync_copy(ones_v, acc.at[dstv.at[i]], add=True)
        return carry

    lax.fori_loop(0, _NCH, body, 0)
    plsc.subcore_barrier()
    pltpu.sync_copy(acc.at[pl.ds(base, _RPT)],
                    out_hbm.at[cid, pl.ds(base, _RPT)])


def _edge_pass(hs_hbm, hs_s, ei_hbm, srcv, dstv, rows, acc, gsems, ssems,
               wid):
    def src_of(b):
        return hs_s if b % 2 == 0 else hs_hbm

    pltpu.sync_copy(ei_hbm.at[0, wid], srcv)
    pltpu.sync_copy(ei_hbm.at[1, wid], dstv)
    for b in range(_NBUF):
        pltpu.async_copy(src_of(b).at[srcv.at[b]], rows.at[b], gsems[b])

    def outer(g, carry):
        for b in range(_NBUF):
            i = g * _NBUF + b
            pltpu.make_async_copy(src_of(b).at[srcv.at[i]], rows.at[b],
                                  gsems[b]).wait()
            pltpu.sync_copy(rows.at[b], acc.at[dstv.at[i]], add=True)
            nxt = i + _NBUF

            @pl.when(nxt < _NCH)
            def _fire():
                pltpu.async_copy(src_of(b).at[srcv.at[nxt]], rows.at[b],
                                 gsems[b])
        return carry

    lax.fori_loop(0, _NCH // _NBUF, outer, 0)


@functools.partial(
    pl.kernel,
    out_type=[
        jax.ShapeDtypeStruct((_NC, _N, _H), jnp.float32),
        jax.ShapeDtypeStruct((_N, _H), jnp.float32),
        jax.ShapeDtypeStruct((_N, _H), jnp.float32),
    ],
    mesh=_mesh,
    scratch_types=[
        pltpu.VMEM((_NCH, _CH), jnp.int32),
        pltpu.VMEM((_NCH, _CH), jnp.int32),
        pltpu.VMEM((_SLOT, _CH, _H), jnp.float32),
        pltpu.VMEM((640, _H), jnp.float32),
        pltpu.VMEM((640, _H), jnp.float32),
        pltpu.VMEM((_RPT, _H), jnp.float32),
        pltpu.VMEM((640, _H), jnp.float32),
        pltpu.VMEM((640, _H), jnp.float32),
        pltpu.VMEM_SHARED((_N, _H), jnp.float32),
        pltpu.VMEM_SHARED((_N, _H), jnp.float32),
    ] + [pltpu.SemaphoreType.DMA] * (2 * _SLOT),
    compiler_params=_scp,
)
def _sc_prop1(h1_hbm, degp_hbm, ei_hbm, out_hbm, dinv_hbm, hs_hbm,
              srcv, dstv, rows, d0v, d1v, h1v, hsv, dvv, hs_s, acc, *sems):
    cid = lax.axis_index("c")
    sid = lax.axis_index("s")
    wid = sid * _NC + cid
    base = sid * _RPT
    pltpu.sync_copy(degp_hbm.at[0, pl.ds(base, _RPT)],
                    d0v.at[pl.ds(0, _RPT)])
    pltpu.sync_copy(degp_hbm.at[1, pl.ds(base, _RPT)],
                    d1v.at[pl.ds(0, _RPT)])
    pltpu.sync_copy(h1_hbm.at[pl.ds(base, _RPT)], h1v)
    def rowfn(r, carry):
        deg = d0v[r, :] + d1v[r, :]
        dinv = _rsqrt16(jnp.maximum(deg, 1.0))
        dvv[r, :] = dinv
        hsv[r, :] = h1v[r, :] * dinv
        return carry

    lax.fori_loop(0, _RPT, rowfn, 0)
    pltpu.sync_copy(hsv.at[pl.ds(0, _RPT)], hs_hbm.at[pl.ds(base, _RPT)])
    pltpu.sync_copy(hsv.at[pl.ds(0, _RPT)], hs_s.at[pl.ds(base, _RPT)])

    @pl.when(cid == 0)
    def _initc0():
        pltpu.sync_copy(hsv.at[pl.ds(0, _RPT)], acc.at[pl.ds(base, _RPT)])
        pltpu.sync_copy(dvv.at[pl.ds(0, _RPT)],
                        dinv_hbm.at[pl.ds(base, _RPT)])

    @pl.when(cid == 1)
    def _initc1():
        def zfn(r, carry):
            d0v[r, :] = jnp.zeros((_H,), jnp.float32)
            return carry

        lax.fori_loop(0, _RPT, zfn, 0)
        pltpu.sync_copy(d0v.at[pl.ds(0, _RPT)], acc.at[pl.ds(base, _RPT)])

    plsc.subcore_barrier()
    _edge_pass(hs_hbm, hs_s, ei_hbm, srcv, dstv, rows, acc,
               sems[:_SLOT], sems[_SLOT:], wid)
    plsc.subcore_barrier()
    pltpu.sync_copy(acc.at[pl.ds(base, _RPT)],
                    out_hbm.at[cid, pl.ds(base, _RPT)])


@functools.partial(
    pl.kernel,
    out_type=[
        jax.ShapeDtypeStruct((_NC, _N, _H), jnp.float32),
        jax.ShapeDtypeStruct((_N, _H), jnp.float32),
    ],
    mesh=_mesh,
    scratch_types=[
        pltpu.VMEM((_NCH, _CH), jnp.int32),
        pltpu.VMEM((_NCH, _CH), jnp.int32),
        pltpu.VMEM((_SLOT, _CH, _H), jnp.float32),
        pltpu.VMEM((_RPT, _H), jnp.float32),
        pltpu.VMEM((_RPT, _H), jnp.float32),
        pltpu.VMEM((_RPT, _H), jnp.float32),
        pltpu.VMEM((_RPT, _H), jnp.float32),
        pltpu.VMEM((1, _H), jnp.float32),
        pltpu.VMEM_SHARED((_N, _H), jnp.float32),
        pltpu.VMEM_SHARED((_N, _H), jnp.float32),
    ] + [pltpu.SemaphoreType.DMA] * (2 * _SLOT),
    compiler_params=_scp,
)
def _sc_prop2(p_hbm, dinv_hbm, ei_hbm, b1_hbm, out_hbm, u_hbm,
              srcv, dstv, rows, p0v, p1v, dvv, hsv, b1v,
              hs_s, acc, *sems):
    cid = lax.axis_index("c")
    sid = lax.axis_index("s")
    wid = sid * _NC + cid
    base = sid * _RPT
    pltpu.sync_copy(p_hbm.at[0, pl.ds(base, _RPT)], p0v)
    pltpu.sync_copy(p_hbm.at[1, pl.ds(base, _RPT)], p1v)
    pltpu.sync_copy(dinv_hbm.at[pl.ds(base, _RPT)], dvv)
    pltpu.sync_copy(b1_hbm, b1v)
    b1 = b1v[0, :]

    def rowfn(r, carry):
        d = dvv[r, :]
        hsv[r, :] = d * jnp.maximum(d * (p0v[r, :] + p1v[r, :]) + b1, 0.0)
        return carry

    lax.fori_loop(0, _RPT, rowfn, 0)
    pltpu.sync_copy(hsv, u_hbm.at[pl.ds(base, _RPT)])
    pltpu.sync_copy(hsv, hs_s.at[pl.ds(base, _RPT)])

    @pl.when(cid == 0)
    def _initc0():
        pltpu.sync_copy(hsv, acc.at[pl.ds(base, _RPT)])

    @pl.when(cid == 1)
    def _initc1():
        def zfn(r, carry):
            p0v[r, :] = jnp.zeros((_H,), jnp.float32)
            return carry

        lax.fori_loop(0, _RPT, zfn, 0)
        pltpu.sync_copy(p0v, acc.at[pl.ds(base, _RPT)])

    plsc.subcore_barrier()
    _edge_pass(u_hbm, hs_s, ei_hbm, srcv, dstv, rows, acc,
               sems[:_SLOT], sems[_SLOT:], wid)
    plsc.subcore_barrier()
    pltpu.sync_copy(acc.at[pl.ds(base, _RPT)],
                    out_hbm.at[cid, pl.ds(base, _RPT)])


_G2 = 2 * _G


@functools.partial(
    pl.kernel,
    out_type=jax.ShapeDtypeStruct((_NS, _G2, _H), jnp.float32),
    mesh=_mesh,
    scratch_types=[
        pltpu.VMEM((640,), jnp.int32),
        pltpu.VMEM((_RPT, _H), jnp.float32),
        pltpu.VMEM((_RPT, _H), jnp.float32),
        pltpu.VMEM((_RPT, _H), jnp.float32),
        pltpu.VMEM((_G2, _H), jnp.float32),
    ],
    compiler_params=_scp,
)
def _sc_pool(p_hbm, dinv_hbm, bat_hbm, out_hbm, batv, p0v, p1v, dvv, gv):
    cid = lax.axis_index("c")
    sid = lax.axis_index("s")

    @pl.when(cid == 0)
    def _core0():
        base = sid * _RPT
        pltpu.sync_copy(p_hbm.at[0, pl.ds(base, _RPT)], p0v)
        pltpu.sync_copy(p_hbm.at[1, pl.ds(base, _RPT)], p1v)
        pltpu.sync_copy(dinv_hbm.at[pl.ds(base, _RPT)], dvv)
        pltpu.sync_copy(bat_hbm.at[sid], batv.at[pl.ds(0, _RPT)])
        z16 = jnp.zeros((_H,), jnp.float32)
        one16 = jnp.ones((_H,), jnp.float32)
        for g in range(_G2):
            gv[g, :] = z16

        def grpfn(gi, carry):
            bvec = batv[pl.ds(gi * 16, 16)]
            for k in range(16):
                r = gi * 16 + k

                @pl.when(r < _RPT)
                def _acc():
                    b = bvec[k]
                    gv[b, :] = gv[b, :] + dvv[r, :] * (p0v[r, :] + p1v[r, :])
                    gv[_G + b, :] = gv[_G + b, :] + one16

            return carry

        lax.fori_loop(0, 40, grpfn, 0)
        pltpu.sync_copy(gv, out_hbm.at[sid])


def _tc1_body(x_ref, w_ref, h_ref):
    h_ref[...] = jnp.dot(x_ref[...], w_ref[...],
                         preferred_element_type=jnp.float32)


_BN = 2000
_GRID = _N // _BN

_tc1 = pl.pallas_call(
    _tc1_body,
    grid=(_GRID,),
    in_specs=[
        pl.BlockSpec((_BN, _FIN), lambda i: (i, 0)),
        pl.BlockSpec((_FIN, _H), lambda i: (0, 0)),
    ],
    out_specs=pl.BlockSpec((_BN, _H), lambda i: (i, 0)),
    out_shape=jax.ShapeDtypeStruct((_N, _H), jnp.float32),
)


def _tc4_body(slab_ref, w2_ref, b2_ref, wl_ref, bl_ref, out_ref):
    red = jnp.sum(slab_ref[...], axis=0)
    mean = red[0:_G] / jnp.maximum(red[_G:_G2], 1.0)
    pooled = jnp.dot(mean, w2_ref[...],
                     preferred_element_type=jnp.float32) + b2_ref[...]
    logits = jnp.dot(pooled, wl_ref[...],
                     preferred_element_type=jnp.float32) + bl_ref[...]
    m = jnp.max(logits, axis=1, keepdims=True)
    lse = jnp.log(jnp.sum(jnp.exp(logits - m), axis=1, keepdims=True)) + m
    out_ref[...] = logits - lse


_tc4 = pl.pallas_call(
    _tc4_body,
    in_specs=[
        pl.BlockSpec((_NS, _G2, _H), lambda: (0, 0, 0)),
        pl.BlockSpec((_H, _H), lambda: (0, 0)),
        pl.BlockSpec((1, _H), lambda: (0, 0)),
        pl.BlockSpec((_H, _C), lambda: (0, 0)),
        pl.BlockSpec((1, _C), lambda: (0, 0)),
    ],
    out_specs=pl.BlockSpec((_G, _C), lambda: (0, 0)),
    out_shape=jax.ShapeDtypeStruct((_G, _C), jnp.float32),
)


def kernel(x, edge_index, batch, W1, b1, W2, b2, Wl, bl):
    ei4 = edge_index.reshape(2, _NW, _NCH, _CH)
    bat2 = batch.reshape(_NS, _RPT)
    degp = _sc_degree(ei4)
    h1 = _tc1(x, W1)
    p1, dinvh, _hs1 = _sc_prop1(h1, degp, ei4)
    p2, _u = _sc_prop2(p1, dinvh, ei4, b1.reshape(1, _H))
    slab = _sc_pool(p2, dinvh, bat2)
    return _tc4(slab, W2, b2.reshape(1, _H), Wl, bl.reshape(1, _C))


# speedup vs baseline: 1.0172x; 1.0172x over previous
import functools

import jax
import jax.numpy as jnp
from jax import lax
from jax.experimental import pallas as pl
from jax.experimental.pallas import tpu as pltpu
from jax.experimental.pallas import tpu_sc as plsc

_N = 10000
_E = 320000
_FIN = 128
_H = 16
_C = 10
_G = 64

_NC = 2
_NS = 16
_NW = _NC * _NS
_EPW = _E // _NW
_CH = 125
_NCH = _EPW // _CH
_NBUF = 8
_SLOT = 16
_NOUT = _NCH // _SLOT
_RPT = _N // _NS

_mesh = plsc.VectorSubcoreMesh(core_axis_name="c", subcore_axis_name="s")
_scp = pltpu.CompilerParams(use_tc_tiling_on_sc=False)


def _rsqrt16(d):
    i = lax.bitcast_convert_type(d, jnp.int32)
    i = jnp.int32(0x5F3759DF) - lax.shift_right_arithmetic(i, 1)
    y = lax.bitcast_convert_type(i, jnp.float32)
    for _ in range(2):
        y = y * (1.5 - 0.5 * d * y * y)
    return y


@functools.partial(
    pl.kernel,
    out_type=jax.ShapeDtypeStruct((_NC, _N, _H), jnp.float32),
    mesh=_mesh,
    scratch_types=[
        pltpu.VMEM((_NCH, _CH), jnp.int32),
        pltpu.VMEM((_CH, _H), jnp.float32),
        pltpu.VMEM_SHARED((_N, _H), jnp.float32),
    ],
    compiler_params=_scp,
)
def _sc_degree(ei_hbm, out_hbm, dstv, ones_v, acc):
    cid = lax.axis_index("c")
    sid = lax.axis_index("s")
    wid = sid * _NC + cid
    pltpu.sync_copy(ei_hbm.at[1, wid], dstv)
    ones16 = jnp.ones((_H,), jnp.float32)
    for r in range(_CH):
        ones_v[r, :] = ones16
    half = jnp.full((_H,), 0.5, jnp.float32)
    for r in range(_CH):
        ones_v[r, :] = half
    base = sid * _RPT
    for k in range(_RPT // _CH):
        pltpu.sync_copy(ones_v, acc.at[pl.ds(base + k * _CH, _CH)])
    for r in range(_CH):
        ones_v[r, :] = ones16
    plsc.subcore_barrier()

    def body(i, carry):
        pltpu.sync_copy(ones_v, acc.at[dstv.at[i]], add=True)
        return carry

    lax.fori_loop(0, _NCH, body, 0)
    plsc.subcore_barrier()
    pltpu.sync_copy(acc.at[pl.ds(base, _RPT)],
                    out_hbm.at[cid, pl.ds(base, _RPT)])


def _edge_pass(hs_hbm, ei_hbm, srcv, dstv, rows, acc, gsems, ssems, wid):
    pltpu.sync_copy(ei_hbm.at[0, wid], srcv)
    pltpu.sync_copy(ei_hbm.at[1, wid], dstv)
    for b in range(_NBUF):
        pltpu.async_copy(hs_hbm.at[srcv.at[b]], rows.at[b], gsems[b])

    def outer(g, carry):
        for b in range(_NBUF):
            i = g * _NBUF + b
            pltpu.make_async_copy(hs_hbm.at[srcv.at[i]], rows.at[b],
                                  gsems[b]).wait()
            pltpu.sync_copy(rows.at[b], acc.at[dstv.at[i]], add=True)
            nxt = i + _NBUF

            @pl.when(nxt < _NCH)
            def _fire():
                pltpu.async_copy(hs_hbm.at[srcv.at[nxt]], rows.at[b],
                                 gsems[b])
        return carry

    lax.fori_loop(0, _NCH // _NBUF, outer, 0)


@functools.partial(
    pl.kernel,
    out_type=[
        jax.ShapeDtypeStruct((_NC, _N, _H), jnp.float32),
        jax.ShapeDtypeStruct((_N, _H), jnp.float32),
        jax.ShapeDtypeStruct((_N, _H), jnp.float32),
    ],
    mesh=_mesh,
    scratch_types=[
        pltpu.VMEM((_NCH, _CH), jnp.int32),
        pltpu.VMEM((_NCH, _CH), jnp.int32),
        pltpu.VMEM((_SLOT, _CH, _H), jnp.float32),
        pltpu.VMEM((640, _H), jnp.float32),
        pltpu.VMEM((640, _H), jnp.float32),
        pltpu.VMEM((_RPT, _H), jnp.float32),
        pltpu.VMEM((640, _H), jnp.float32),
        pltpu.VMEM((640, _H), jnp.float32),
        pltpu.VMEM_SHARED((_N, _H), jnp.float32),
    ] + [pltpu.SemaphoreType.DMA] * (2 * _SLOT),
    compiler_params=_scp,
)
def _sc_prop1(h1_hbm, degp_hbm, ei_hbm, out_hbm, dinv_hbm, hs_hbm,
              srcv, dstv, rows, d0v, d1v, h1v, hsv, dvv, acc, *sems):
    cid = lax.axis_index("c")
    sid = lax.axis_index("s")
    wid = sid * _NC + cid
    base = sid * _RPT
    pltpu.sync_copy(degp_hbm.at[0, pl.ds(base, _RPT)],
                    d0v.at[pl.ds(0, _RPT)])
    pltpu.sync_copy(degp_hbm.at[1, pl.ds(base, _RPT)],
                    d1v.at[pl.ds(0, _RPT)])
    pltpu.sync_copy(h1_hbm.at[pl.ds(base, _RPT)], h1v)
    def rowfn(r, carry):
        deg = d0v[r, :] + d1v[r, :]
        dinv = _rsqrt16(jnp.maximum(deg, 1.0))
        dvv[r, :] = dinv
        hsv[r, :] = h1v[r, :] * dinv
        return carry

    lax.fori_loop(0, _RPT, rowfn, 0)
    pltpu.sync_copy(hsv.at[pl.ds(0, _RPT)], hs_hbm.at[pl.ds(base, _RPT)])

    @pl.when(cid == 0)
    def _initc0():
        pltpu.sync_copy(hsv.at[pl.ds(0, _RPT)], acc.at[pl.ds(base, _RPT)])
        pltpu.sync_copy(dvv.at[pl.ds(0, _RPT)],
                        dinv_hbm.at[pl.ds(base, _RPT)])

    @pl.when(cid == 1)
    def _initc1():
        def zfn(r, carry):
            d0v[r, :] = jnp.zeros((_H,), jnp.float32)
            return carry

        lax.fori_loop(0, _RPT, zfn, 0)
        pltpu.sync_copy(d0v.at[pl.ds(0, _RPT)], acc.at[pl.ds(base, _RPT)])

    plsc.subcore_barrier()
    _edge_pass(hs_hbm, ei_hbm, srcv, dstv, rows, acc,
               sems[:_SLOT], sems[_SLOT:], wid)
    plsc.subcore_barrier()
    pltpu.sync_copy(acc.at[pl.ds(base, _RPT)],
                    out_hbm.at[cid, pl.ds(base, _RPT)])


@functools.partial(
    pl.kernel,
    out_type=[
        jax.ShapeDtypeStruct((_NC, _N, _H), jnp.float32),
        jax.ShapeDtypeStruct((_N, _H), jnp.float32),
    ],
    mesh=_mesh,
    scratch_types=[
        pltpu.VMEM((_NCH, _CH), jnp.int32),
        pltpu.VMEM((_NCH, _CH), jnp.int32),
        pltpu.VMEM((_SLOT, _CH, _H), jnp.float32),
        pltpu.VMEM((_RPT, _H), jnp.float32),
        pltpu.VMEM((_RPT, _H), jnp.float32),
        pltpu.VMEM((_RPT, _H), jnp.float32),
        pltpu.VMEM((_RPT, _H), jnp.float32),
        pltpu.VMEM((1, _H), jnp.float32),
        pltpu.VMEM_SHARED((_N, _H), jnp.float32),
    ] + [pltpu.SemaphoreType.DMA] * (2 * _SLOT),
    compiler_params=_scp,
)
def _sc_prop2(p_hbm, dinv_hbm, ei_hbm, b1_hbm, out_hbm, u_hbm,
              srcv, dstv, rows, p0v, p1v, dvv, hsv, b1v,
              acc, *sems):
    cid = lax.axis_index("c")
    sid = lax.axis_index("s")
    wid = sid * _NC + cid
    base = sid * _RPT
    pltpu.sync_copy(p_hbm.at[0, pl.ds(base, _RPT)], p0v)
    pltpu.sync_copy(p_hbm.at[1, pl.ds(base, _RPT)], p1v)
    pltpu.sync_copy(dinv_hbm.at[pl.ds(base, _RPT)], dvv)
    pltpu.sync_copy(b1_hbm, b1v)
    b1 = b1v[0, :]

    def rowfn(r, carry):
        d = dvv[r, :]
        hsv[r, :] = d * jnp.maximum(d * (p0v[r, :] + p1v[r, :]) + b1, 0.0)
        return carry

    lax.fori_loop(0, _RPT, rowfn, 0)
    pltpu.sync_copy(hsv, u_hbm.at[pl.ds(base, _RPT)])

    @pl.when(cid == 0)
    def _initc0():
        pltpu.sync_copy(hsv, acc.at[pl.ds(base, _RPT)])

    @pl.when(cid == 1)
    def _initc1():
        def zfn(r, carry):
            p0v[r, :] = jnp.zeros((_H,), jnp.float32)
            return carry

        lax.fori_loop(0, _RPT, zfn, 0)
        pltpu.sync_copy(p0v, acc.at[pl.ds(base, _RPT)])

    plsc.subcore_barrier()
    _edge_pass(u_hbm, ei_hbm, srcv, dstv, rows, acc,
               sems[:_SLOT], sems[_SLOT:], wid)
    plsc.subcore_barrier()
    pltpu.sync_copy(acc.at[pl.ds(base, _RPT)],
                    out_hbm.at[cid, pl.ds(base, _RPT)])


_G2 = 2 * _G


@functools.partial(
    pl.kernel,
    out_type=jax.ShapeDtypeStruct((_NS, _G2, _H), jnp.float32),
    mesh=_mesh,
    scratch_types=[
        pltpu.VMEM((640,), jnp.int32),
        pltpu.VMEM((_RPT, _H), jnp.float32),
        pltpu.VMEM((_RPT, _H), jnp.float32),
        pltpu.VMEM((_RPT, _H), jnp.float32),
        pltpu.VMEM((_G2, _H), jnp.float32),
    ],
    compiler_params=_scp,
)
def _sc_pool(p_hbm, dinv_hbm, bat_hbm, out_hbm, batv, p0v, p1v, dvv, gv):
    cid = lax.axis_index("c")
    sid = lax.axis_index("s")

    @pl.when(cid == 0)
    def _core0():
        base = sid * _RPT
        pltpu.sync_copy(p_hbm.at[0, pl.ds(base, _RPT)], p0v)
        pltpu.sync_copy(p_hbm.at[1, pl.ds(base, _RPT)], p1v)
        pltpu.sync_copy(dinv_hbm.at[pl.ds(base, _RPT)], dvv)
        pltpu.sync_copy(bat_hbm.at[sid], batv.at[pl.ds(0, _RPT)])
        z16 = jnp.zeros((_H,), jnp.float32)
        one16 = jnp.ones((_H,), jnp.float32)
        for g in range(_G2):
            gv[g, :] = z16

        def grpfn(gi, carry):
            bvec = batv[pl.ds(gi * 16, 16)]
            for k in range(16):
                r = gi * 16 + k

                @pl.when(r < _RPT)
                def _acc():
                    b = bvec[k]
                    gv[b, :] = gv[b, :] + dvv[r, :] * (p0v[r, :] + p1v[r, :])
                    gv[_G + b, :] = gv[_G + b, :] + one16

            return carry

        lax.fori_loop(0, 40, grpfn, 0)
        pltpu.sync_copy(gv, out_hbm.at[sid])


def _tc1_body(x_ref, w_ref, h_ref):
    h_ref[...] = jnp.dot(x_ref[...], w_ref[...],
                         preferred_element_type=jnp.float32)


_BN = 2000
_GRID = _N // _BN

_tc1 = pl.pallas_call(
    _tc1_body,
    grid=(_GRID,),
    in_specs=[
        pl.BlockSpec((_BN, _FIN), lambda i: (i, 0)),
        pl.BlockSpec((_FIN, _H), lambda i: (0, 0)),
    ],
    out_specs=pl.BlockSpec((_BN, _H), lambda i: (i, 0)),
    out_shape=jax.ShapeDtypeStruct((_N, _H), jnp.float32),
)


def _tc4_body(slab_ref, w2_ref, b2_ref, wl_ref, bl_ref, out_ref):
    red = jnp.sum(slab_ref[...], axis=0)
    mean = red[0:_G] / jnp.maximum(red[_G:_G2], 1.0)
    pooled = jnp.dot(mean, w2_ref[...],
                     preferred_element_type=jnp.float32) + b2_ref[...]
    logits = jnp.dot(pooled, wl_ref[...],
                     preferred_element_type=jnp.float32) + bl_ref[...]
    m = jnp.max(logits, axis=1, keepdims=True)
    lse = jnp.log(jnp.sum(jnp.exp(logits - m), axis=1, keepdims=True)) + m
    out_ref[...] = logits - lse


_tc4 = pl.pallas_call(
    _tc4_body,
    in_specs=[
        pl.BlockSpec((_NS, _G2, _H), lambda: (0, 0, 0)),
        pl.BlockSpec((_H, _H), lambda: (0, 0)),
        pl.BlockSpec((1, _H), lambda: (0, 0)),
        pl.BlockSpec((_H, _C), lambda: (0, 0)),
        pl.BlockSpec((1, _C), lambda: (0, 0)),
    ],
    out_specs=pl.BlockSpec((_G, _C), lambda: (0, 0)),
    out_shape=jax.ShapeDtypeStruct((_G, _C), jnp.float32),
)


def kernel(x, edge_index, batch, W1, b1, W2, b2, Wl, bl):
    ei4 = edge_index.reshape(2, _NW, _NCH, _CH)
    bat2 = batch.reshape(_NS, _RPT)
    degp = _sc_degree(ei4)
    h1 = _tc1(x, W1)
    p1, dinvh, _hs1 = _sc_prop1(h1, degp, ei4)
    p2, _u = _sc_prop2(p1, dinvh, ei4, b1.reshape(1, _H))
    slab = _sc_pool(p2, dinvh, bat2)
    return _tc4(slab, W2, b2.reshape(1, _H), Wl, bl.reshape(1, _C))
